# Initial kernel scaffold; baseline (speedup 1.0000x reference)
import jax
import jax.numpy as jnp
from jax.experimental import pallas as pl


def kernel(init_pos_ids, hop_dis_ids, time_dis_ids, type_dis_ids, pos_table, hop_table, time_table, type_table, gamma, beta):
    raise NotImplementedError("write your pallas kernel here")


# speedup vs baseline: 2.1162x; 2.1162x over previous
import functools

import jax
import jax.numpy as jnp
from jax import lax
from jax.experimental import pallas as pl
from jax.experimental.pallas import tpu as pltpu
from jax.experimental.pallas import tpu_sc as plsc

_EPS = 1e-12
_H = 64
_POS_ROWS = 100000
_SMALL_ROWS = 2000
_BLK = 2000


def _ln_body(pos_ref, small_ref, g_ref, b_ref, o_ref):
    i = pl.program_id(0)
    last = pl.num_programs(0) - 1
    x = jnp.where(i == last, small_ref[...], pos_ref[...])
    m = jnp.mean(x, axis=-1, keepdims=True)
    v = jnp.mean((x - m) ** 2, axis=-1, keepdims=True)
    o_ref[...] = (x - m) / jnp.sqrt(v + _EPS) * g_ref[...] + b_ref[...]


def _ln_combined(pos_table, smalls, gamma2d, beta2d):
    npos = pos_table.shape[0]
    grid = npos // _BLK + 1
    return pl.pallas_call(
        _ln_body,
        grid=(grid,),
        in_specs=[
            pl.BlockSpec((_BLK, _H), lambda i: (jnp.minimum(i, grid - 2), 0)),
            pl.BlockSpec((_SMALL_ROWS, _H), lambda i: (0, 0)),
            pl.BlockSpec((1, _H), lambda i: (0, 0)),
            pl.BlockSpec((1, _H), lambda i: (0, 0)),
        ],
        out_specs=pl.BlockSpec((_BLK, _H), lambda i: (i, 0)),
        out_shape=jax.ShapeDtypeStruct((npos + _SMALL_ROWS, _H), jnp.float32),
    )(pos_table, smalls, gamma2d, beta2d)


def _ilv_body(p_ref, h_ref, t_ref, y_ref, o_ref):
    o_ref[...] = jnp.concatenate(
        [
            p_ref[...],
            h_ref[...] + _POS_ROWS,
            t_ref[...] + (_POS_ROWS + 100),
            y_ref[...] + (_POS_ROWS + 116),
        ],
        axis=1,
    )


def _interleave_ids(ids, n_flat):
    blk = 2048
    cols = [a.reshape(n_flat, 1) for a in ids]
    spec = pl.BlockSpec((blk, 1), lambda i: (i, 0))
    return pl.pallas_call(
        _ilv_body,
        grid=(n_flat // blk,),
        in_specs=[spec, spec, spec, spec],
        out_specs=pl.BlockSpec((blk, 4), lambda i: (i, 0)),
        out_shape=jax.ShapeDtypeStruct((n_flat, 4), jnp.int32),
    )(*cols).reshape(n_flat * 4)


def _sc_gather(table, midx, sub):
    n = midx.shape[0]
    info = plsc.get_sparse_core_info()
    nw = info.num_cores * info.num_subcores
    assert n % (nw * sub) == 0
    chunk = n // nw
    nsub = chunk // sub

    mesh = plsc.VectorSubcoreMesh(core_axis_name="c", subcore_axis_name="s")

    @functools.partial(
        pl.kernel,
        out_type=jax.ShapeDtypeStruct((n, _H), jnp.float32),
        mesh=mesh,
        compiler_params=pltpu.CompilerParams(use_tc_tiling_on_sc=False),
        scratch_types=[
            pltpu.VMEM((chunk,), jnp.int32),
            pltpu.VMEM((sub, _H), jnp.float32),
            pltpu.SemaphoreType.DMA,
        ],
    )
    def k(tab_h, midx_h, out_h, idx_v, rows_v, sem):
        wid = lax.axis_index("s") * info.num_cores + lax.axis_index("c")
        base = wid * chunk
        pltpu.sync_copy(midx_h.at[pl.ds(base, chunk)], idx_v)

        def body(j, carry):
            pltpu.async_copy(
                tab_h.at[idx_v.at[pl.ds(j * sub, sub)]], rows_v, sem
            ).wait()
            pltpu.sync_copy(rows_v, out_h.at[pl.ds(base + j * sub, sub)])
            return carry

        lax.fori_loop(0, nsub, body, 0)

    return k(table, midx)


def kernel(init_pos_ids, hop_dis_ids, time_dis_ids, type_dis_ids,
           pos_table, hop_table, time_table, type_table, gamma, beta):
    g2 = gamma.reshape(1, _H)
    b2 = beta.reshape(1, _H)

    smalls = (
        jnp.zeros((_SMALL_ROWS, _H), jnp.float32)
        .at[0:100].set(hop_table)
        .at[100:116].set(time_table)
        .at[116:124].set(type_table)
    )
    table_n = _ln_combined(pos_table, smalls, g2, b2)

    n_flat = init_pos_ids.size
    midx = _interleave_ids(
        (init_pos_ids, hop_dis_ids, time_dis_ids, type_dis_ids), n_flat
    )

    out2d = _sc_gather(table_n, midx, sub=128)

    s = init_pos_ids.shape
    return out2d.reshape(s[0], s[1], s[2], 4, _H)


# speedup vs baseline: 2.1246x; 1.0040x over previous
import functools

import jax
import jax.numpy as jnp
from jax import lax
from jax.experimental import pallas as pl
from jax.experimental.pallas import tpu as pltpu
from jax.experimental.pallas import tpu_sc as plsc

_EPS = 1e-12
_H = 64
_POS_ROWS = 100000
_SMALL_ROWS = 2000
_BLK = 2000


def _ln_body(pos_ref, small_ref, g_ref, b_ref, o_ref):
    i = pl.program_id(0)
    last = pl.num_programs(0) - 1
    x = jnp.where(i == last, small_ref[...], pos_ref[...])
    m = jnp.mean(x, axis=-1, keepdims=True)
    v = jnp.mean((x - m) ** 2, axis=-1, keepdims=True)
    o_ref[...] = (x - m) / jnp.sqrt(v + _EPS) * g_ref[...] + b_ref[...]


def _ln_combined(pos_table, smalls, gamma2d, beta2d):
    npos = pos_table.shape[0]
    grid = npos // _BLK + 1
    return pl.pallas_call(
        _ln_body,
        grid=(grid,),
        in_specs=[
            pl.BlockSpec((_BLK, _H), lambda i: (jnp.minimum(i, grid - 2), 0)),
            pl.BlockSpec((_SMALL_ROWS, _H), lambda i: (0, 0)),
            pl.BlockSpec((1, _H), lambda i: (0, 0)),
            pl.BlockSpec((1, _H), lambda i: (0, 0)),
        ],
        out_specs=pl.BlockSpec((_BLK, _H), lambda i: (i, 0)),
        out_shape=jax.ShapeDtypeStruct((npos + _SMALL_ROWS, _H), jnp.float32),
    )(pos_table, smalls, gamma2d, beta2d)


def _ilv_body(p_ref, h_ref, t_ref, y_ref, o_ref):
    o_ref[...] = jnp.concatenate(
        [
            p_ref[...],
            h_ref[...] + _POS_ROWS,
            t_ref[...] + (_POS_ROWS + 100),
            y_ref[...] + (_POS_ROWS + 116),
        ],
        axis=1,
    )


def _interleave_ids(ids, n_flat):
    blk = 2048
    cols = [a.reshape(n_flat, 1) for a in ids]
    spec = pl.BlockSpec((blk, 1), lambda i: (i, 0))
    return pl.pallas_call(
        _ilv_body,
        grid=(n_flat // blk,),
        in_specs=[spec, spec, spec, spec],
        out_specs=pl.BlockSpec((blk, 4), lambda i: (i, 0)),
        out_shape=jax.ShapeDtypeStruct((n_flat, 4), jnp.int32),
    )(*cols).reshape(n_flat * 4)


def _sc_gather(table, midx, sub, nbuf):
    n = midx.shape[0]
    info = plsc.get_sparse_core_info()
    nw = info.num_cores * info.num_subcores
    assert n % (nw * sub * nbuf) == 0
    chunk = n // nw
    nsub = chunk // sub
    ngroups = nsub // nbuf

    mesh = plsc.VectorSubcoreMesh(core_axis_name="c", subcore_axis_name="s")

    @functools.partial(
        pl.kernel,
        out_type=jax.ShapeDtypeStruct((n, _H), jnp.float32),
        mesh=mesh,
        compiler_params=pltpu.CompilerParams(use_tc_tiling_on_sc=False),
        scratch_types=[
            pltpu.VMEM((chunk,), jnp.int32),
            pltpu.VMEM((nbuf, sub, _H), jnp.float32),
            pltpu.SemaphoreType.DMA((nbuf,)),
            pltpu.SemaphoreType.DMA((nbuf,)),
        ],
    )
    def k(tab_h, midx_h, out_h, idx_v, rows_v, gsem, wsem):
        wid = lax.axis_index("s") * info.num_cores + lax.axis_index("c")
        base = wid * chunk
        pltpu.sync_copy(midx_h.at[pl.ds(base, chunk)], idx_v)

        def g_copy(b, j):
            return pltpu.make_async_copy(
                tab_h.at[idx_v.at[pl.ds(j * sub, sub)]],
                rows_v.at[b],
                gsem.at[b],
            )

        def w_copy(b, j):
            return pltpu.make_async_copy(
                rows_v.at[b],
                out_h.at[pl.ds(base + j * sub, sub)],
                wsem.at[b],
            )

        for b in range(nbuf):
            g_copy(b, b).start()

        def group(g, carry):
            j0 = g * nbuf
            for b in range(nbuf):
                g_copy(b, j0 + b).wait()
                w_copy(b, j0 + b).start()
            for b in range(nbuf):
                w_copy(b, j0 + b).wait()
                g_copy(b, j0 + b + nbuf).start()
            return carry

        lax.fori_loop(0, ngroups - 1, group, 0)

        j0 = (ngroups - 1) * nbuf
        for b in range(nbuf):
            g_copy(b, j0 + b).wait()
            w_copy(b, j0 + b).start()
        for b in range(nbuf):
            w_copy(b, j0 + b).wait()

    return k(table, midx)


def kernel(init_pos_ids, hop_dis_ids, time_dis_ids, type_dis_ids,
           pos_table, hop_table, time_table, type_table, gamma, beta):
    g2 = gamma.reshape(1, _H)
    b2 = beta.reshape(1, _H)

    smalls = (
        jnp.zeros((_SMALL_ROWS, _H), jnp.float32)
        .at[0:100].set(hop_table)
        .at[100:116].set(time_table)
        .at[116:124].set(type_table)
    )
    table_n = _ln_combined(pos_table, smalls, g2, b2)

    n_flat = init_pos_ids.size
    midx = _interleave_ids(
        (init_pos_ids, hop_dis_ids, time_dis_ids, type_dis_ids), n_flat
    )

    out2d = _sc_gather(table_n, midx, sub=128, nbuf=4)

    s = init_pos_ids.shape
    return out2d.reshape(s[0], s[1], s[2], 4, _H)
